# baseline (speedup 1.0000x reference)
import functools

import jax
import jax.numpy as jnp
from jax import lax
from jax.experimental import pallas as pl
from jax.experimental.pallas import tpu as pltpu
from jax.experimental.pallas import tpu_sc as plsc

U_DIM = 64
I_DIM = 64


def _sc_gather(user_table, user_id, item_table, item_id):
    B = user_id.shape[0]
    info = plsc.get_sparse_core_info()
    NC, NS = info.num_cores, info.num_subcores
    NW = NC * NS
    b_per_w = B // NW
    mesh = plsc.VectorSubcoreMesh(core_axis_name="c", subcore_axis_name="s")

    @functools.partial(
        pl.kernel,
        mesh=mesh,
        compiler_params=pltpu.CompilerParams(use_tc_tiling_on_sc=False),
        out_type=(
            jax.ShapeDtypeStruct((B, U_DIM), jnp.float32),
            jax.ShapeDtypeStruct((B, I_DIM), jnp.float32),
        ),
        scratch_types=[
            pltpu.VMEM((b_per_w,), jnp.int32),
            pltpu.VMEM((b_per_w,), jnp.int32),
            pltpu.VMEM((b_per_w, U_DIM), jnp.float32),
            pltpu.VMEM((b_per_w, I_DIM), jnp.float32),
            pltpu.SemaphoreType.DMA,
            pltpu.SemaphoreType.DMA,
        ],
    )
    def gather_k(ut_hbm, uid_hbm, it_hbm, iid_hbm, out_u_hbm, out_i_hbm,
                 uidx_v, iidx_v, urows_v, irows_v, sem_u, sem_i):
        wid = lax.axis_index("s") * NC + lax.axis_index("c")
        base = wid * b_per_w
        pltpu.sync_copy(uid_hbm.at[pl.ds(base, b_per_w)], uidx_v)
        pltpu.sync_copy(iid_hbm.at[pl.ds(base, b_per_w)], iidx_v)
        cu = pltpu.async_copy(ut_hbm.at[uidx_v], urows_v, sem_u)
        ci = pltpu.async_copy(it_hbm.at[iidx_v], irows_v, sem_i)
        cu.wait()
        pltpu.sync_copy(urows_v, out_u_hbm.at[pl.ds(base, b_per_w)])
        ci.wait()
        pltpu.sync_copy(irows_v, out_i_hbm.at[pl.ds(base, b_per_w)])

    return gather_k(user_table, user_id, item_table, item_id)


def _tc_combine(ue, uf, ie, W_uf, b_uf, W_final, b_final):
    B, ufd = uf.shape
    blk = 2048
    n_tasks = W_final.shape[1]

    def body(ue_ref, uf_ref, ie_ref, wuf_ref, buf_ref, wf_ref, bf_ref, out_ref):
        wf = wf_ref[...]
        t = jnp.dot(uf_ref[...], wuf_ref[...],
                    preferred_element_type=jnp.float32) + buf_ref[...]
        acc = jnp.dot(ue_ref[...], wf[0:U_DIM, :],
                      preferred_element_type=jnp.float32)
        acc += jnp.dot(t, wf[U_DIM:2 * U_DIM, :],
                       preferred_element_type=jnp.float32)
        acc += jnp.dot(ie_ref[...], wf[2 * U_DIM:, :],
                       preferred_element_type=jnp.float32)
        out_ref[...] = acc + bf_ref[...]

    return pl.pallas_call(
        body,
        grid=(B // blk,),
        in_specs=[
            pl.BlockSpec((blk, U_DIM), lambda i: (i, 0)),
            pl.BlockSpec((blk, ufd), lambda i: (i, 0)),
            pl.BlockSpec((blk, I_DIM), lambda i: (i, 0)),
            pl.BlockSpec((ufd, U_DIM), lambda i: (0, 0)),
            pl.BlockSpec((1, U_DIM), lambda i: (0, 0)),
            pl.BlockSpec((2 * U_DIM + I_DIM, n_tasks), lambda i: (0, 0)),
            pl.BlockSpec((1, n_tasks), lambda i: (0, 0)),
        ],
        out_specs=pl.BlockSpec((blk, n_tasks), lambda i: (i, 0)),
        out_shape=jax.ShapeDtypeStruct((B, n_tasks), jnp.float32),
    )(ue, uf, ie, W_uf, b_uf, W_final, b_final)


def kernel(user_id, user_features, item_id, user_table, item_table,
           W_uf, b_uf, W_final, b_final):
    ue, ie = _sc_gather(user_table, user_id.astype(jnp.int32),
                        item_table, item_id.astype(jnp.int32))
    return _tc_combine(ue, user_features, ie, W_uf,
                       b_uf.reshape(1, -1), W_final, b_final.reshape(1, -1))
